# baseline (speedup 1.0000x reference)
import functools

import jax
import jax.numpy as jnp
from jax import lax
from jax.experimental import pallas as pl
from jax.experimental.pallas import tpu as pltpu


_TOK_BLK = 512


def _argmin_body(x_ref, cb_ref, idx_ref):
    xb = x_ref[...]
    cb = cb_ref[...]
    cross = lax.dot_general(
        xb, cb, (((1,), (1,)), ((), ())),
        preferred_element_type=jnp.float32)
    x_sq = jnp.sum(xb * xb, axis=1, keepdims=True)
    c_sq = jnp.sum(cb * cb, axis=1)
    d2 = (x_sq + c_sq[None, :]) - 2.0 * cross
    dist = jnp.sqrt(jnp.clip(d2, 0.0, None))
    dmin = jnp.min(dist, axis=1, keepdims=True)
    k = dist.shape[1]
    kiota = lax.broadcasted_iota(jnp.int32, dist.shape, 1)
    idx = jnp.min(jnp.where(dist == dmin, kiota, k), axis=1)
    idx_ref[0, 0, :] = idx


def _nearest_indices(x2d, codebook):
    n_tok = x2d.shape[0]
    grid = n_tok // _TOK_BLK
    out = pl.pallas_call(
        _argmin_body,
        grid=(grid,),
        in_specs=[
            pl.BlockSpec((_TOK_BLK, x2d.shape[1]), lambda i: (i, 0)),
            pl.BlockSpec(codebook.shape, lambda i: (0, 0)),
        ],
        out_specs=pl.BlockSpec((1, 1, _TOK_BLK), lambda i: (i, 0, 0)),
        out_shape=jax.ShapeDtypeStruct((grid, 1, _TOK_BLK), jnp.int32),
    )(x2d, codebook)
    return out.reshape(n_tok)


def kernel(x, codebook, embedding):
    b, t, d = x.shape
    x2d = x.reshape(b * t, d)
    idx = _nearest_indices(x2d, codebook)
    out = jnp.take(embedding, idx, axis=0)
    return out.reshape(b, t, d)
